# Initial kernel scaffold; baseline (speedup 1.0000x reference)
import jax
import jax.numpy as jnp
from jax.experimental import pallas as pl


def kernel(x, table):
    raise NotImplementedError("write your pallas kernel here")


# speedup vs baseline: 4.8096x; 4.8096x over previous
import functools

import jax
import jax.numpy as jnp
from jax import lax
from jax.experimental import pallas as pl
from jax.experimental.pallas import tpu as pltpu
from jax.experimental.pallas import tpu_sc as plsc

_CHUNK = 1024


@functools.lru_cache(maxsize=None)
def _make_gather(n, d):
    info = plsc.get_sparse_core_info()
    nc, ns = info.num_cores, info.num_subcores
    nw = nc * ns
    assert n % nw == 0
    per_w = n // nw
    c = min(_CHUNK, per_w)
    assert per_w % c == 0 and c % 8 == 0
    n_steps = per_w // c

    mesh = plsc.VectorSubcoreMesh(core_axis_name="c", subcore_axis_name="s")

    @functools.partial(
        pl.kernel,
        mesh=mesh,
        out_type=jax.ShapeDtypeStruct((n, d), jnp.float32),
        compiler_params=pltpu.CompilerParams(use_tc_tiling_on_sc=False),
        scratch_types=[
            pltpu.VMEM((c,), jnp.int32),
            pltpu.VMEM((c, d), jnp.float32),
            pltpu.SemaphoreType.DMA,
        ],
    )
    def gather_kernel(idx_hbm, table_hbm, out_hbm, idx_v, rows_v, sem):
        wid = lax.axis_index("s") * nc + lax.axis_index("c")
        base = wid * per_w

        def step(i, carry):
            off = pl.multiple_of(base + i * c, 8)
            pltpu.sync_copy(idx_hbm.at[pl.ds(off, c)], idx_v)
            pltpu.async_copy(table_hbm.at[idx_v], rows_v, sem).wait()
            pltpu.sync_copy(rows_v, out_hbm.at[pl.ds(off, c)])
            return carry

        lax.fori_loop(0, n_steps, step, 0)

    return gather_kernel


def kernel(x, table):
    b, h = x.shape
    _, d = table.shape
    n = b * h
    idx = x.reshape(n).astype(jnp.int32)
    out = _make_gather(n, d)(idx, table)
    return out.reshape(b, h, d)


# speedup vs baseline: 4.9548x; 1.0302x over previous
import functools

import jax
import jax.numpy as jnp
from jax import lax
from jax.experimental import pallas as pl
from jax.experimental.pallas import tpu as pltpu
from jax.experimental.pallas import tpu_sc as plsc

_CHUNK = 1024
_NBUF = 2


@functools.lru_cache(maxsize=None)
def _make_gather(n, d):
    info = plsc.get_sparse_core_info()
    nc, ns = info.num_cores, info.num_subcores
    nw = nc * ns
    assert n % nw == 0
    per_w = n // nw
    c = min(_CHUNK, per_w)
    nb = _NBUF
    assert per_w % c == 0 and c % 8 == 0
    n_steps = per_w // c
    assert n_steps % nb == 0 and n_steps >= 2 * nb

    mesh = plsc.VectorSubcoreMesh(core_axis_name="c", subcore_axis_name="s")

    @functools.partial(
        pl.kernel,
        mesh=mesh,
        out_type=jax.ShapeDtypeStruct((n, d), jnp.float32),
        compiler_params=pltpu.CompilerParams(use_tc_tiling_on_sc=False),
        scratch_types=[
            pltpu.VMEM((nb, c), jnp.int32),
            pltpu.VMEM((nb, c, d), jnp.float32),
        ] + [pltpu.SemaphoreType.DMA] * (2 * nb),
    )
    def gather_kernel(idx_hbm, table_hbm, out_hbm, idx_v, rows_v, *sems):
        sem_g, sem_s = sems[:nb], sems[nb:]
        wid = lax.axis_index("s") * nc + lax.axis_index("c")
        base = wid * per_w

        def chunk_off(i):
            return pl.multiple_of(base + i * c, 8)

        def issue_gather(i, b):
            pltpu.sync_copy(idx_hbm.at[pl.ds(chunk_off(i), c)], idx_v.at[b])
            pltpu.async_copy(table_hbm.at[idx_v.at[b]], rows_v.at[b], sem_g[b])

        def wait_gather(b):
            pltpu.make_async_copy(
                table_hbm.at[idx_v.at[b]], rows_v.at[b], sem_g[b]).wait()

        def issue_store(i, b):
            pltpu.async_copy(rows_v.at[b], out_hbm.at[pl.ds(chunk_off(i), c)],
                             sem_s[b])

        def wait_store(i, b):
            pltpu.make_async_copy(
                rows_v.at[b], out_hbm.at[pl.ds(chunk_off(i), c)],
                sem_s[b]).wait()

        for b in range(nb):
            issue_gather(b, b)

        def outer(jo, carry):
            for b in range(nb):
                i = jo * nb + b
                wait_gather(b)
                issue_store(i, b)
                wait_store(i, b)
                issue_gather(i + nb, b)
            return carry

        lax.fori_loop(0, n_steps // nb - 1, outer, 0, unroll=False)

        for b in range(nb):
            i = n_steps - nb + b
            wait_gather(b)
            issue_store(i, b)
            wait_store(i, b)

    return gather_kernel


def kernel(x, table):
    b, h = x.shape
    _, d = table.shape
    n = b * h
    idx = x.reshape(n).astype(jnp.int32)
    out = _make_gather(n, d)(idx, table)
    return out.reshape(b, h, d)
